# baseline (speedup 1.0000x reference)
import functools

import jax
import jax.numpy as jnp
from jax.experimental import pallas as pl
from jax.experimental.pallas import tpu as pltpu

_BR = 2000
_S = 5
_SR = _BR // _S


def _sub_specs(b):
    return [
        pl.BlockSpec((_SR, b), functools.partial(lambda j, k: (_S * k + j, 0), j))
        for j in range(_S)
    ]


def _p1(*refs, nu_blocks):
    a_refs = refs[:_S]
    u_ref, p_ref, gu_ref, gp_ref, m1t_ref = refs[_S:]
    k = pl.program_id(0)

    @pl.when(k == 0)
    def _():
        m1t_ref[...] = jnp.zeros_like(m1t_ref)

    is_user = k < nu_blocks
    e0 = jnp.where(is_user, u_ref[...], p_ref[...])
    g = jnp.where(is_user, gu_ref[0, 0], gp_ref[0, 0])
    w = (g * e0).astype(jnp.bfloat16)
    acc = None
    for j in range(_S):
        prod = jax.lax.dot_general(
            w[j * _SR:(j + 1) * _SR], a_refs[j][...], (((0,), (0,)), ((), ())),
            preferred_element_type=jnp.float32)
        acc = prod if acc is None else acc + prod
    m1t_ref[...] += acc


def _p2(*refs, nu_blocks):
    a_refs = refs[:_S]
    m1t_ref, gu_ref, gp_ref, e1_ref, m2t_ref, y1_scr = refs[_S:]
    k = pl.program_id(0)

    @pl.when(k == 0)
    def _():
        m2t_ref[...] = jnp.zeros_like(m2t_ref)
        y1_scr[...] = m1t_ref[...].astype(jnp.bfloat16).T

    g = jnp.where(k < nu_blocks, gu_ref[0, 0], gp_ref[0, 0])
    y1 = y1_scr[...]
    acc = None
    for j in range(_S):
        a = a_refs[j][...]
        t = jax.lax.dot_general(
            a, y1, (((1,), (0,)), ((), ())), preferred_element_type=jnp.float32)
        e1_ref[j * _SR:(j + 1) * _SR] = t
        x = (g * t).astype(jnp.bfloat16)
        prod = jax.lax.dot_general(
            x, a, (((0,), (0,)), ((), ())), preferred_element_type=jnp.float32)
        acc = prod if acc is None else acc + prod
    m2t_ref[...] += acc


def _p3(*refs, nu_blocks):
    a_refs = refs[:_S]
    m2t_ref, u_ref, p_ref, e1_ref, uo_ref, po_ref, y2_scr = refs[_S:]
    k = pl.program_id(0)

    @pl.when(k == 0)
    def _():
        y2_scr[...] = m2t_ref[...].astype(jnp.bfloat16).T

    is_user = k < nu_blocks
    e0 = jnp.where(is_user, u_ref[...], p_ref[...])
    y2 = y2_scr[...]
    for j in range(_S):
        t = jax.lax.dot_general(
            a_refs[j][...], y2, (((1,), (0,)), ((), ())),
            preferred_element_type=jnp.float32)
        sl = slice(j * _SR, (j + 1) * _SR)
        res = (e0[sl] + e1_ref[sl] + t) * (1.0 / 3.0)

        @pl.when(is_user)
        def _(res=res, sl=sl):
            uo_ref[sl] = res

        @pl.when(jnp.logical_not(is_user))
        def _(res=res, sl=sl):
            po_ref[sl] = res


def kernel(users_embedding, product_embedding, adj_matrix, degreeV_matrix,
           degreeE_matrix, gate_user, gate_product):
    nu, dim = users_embedding.shape
    npr = product_embedding.shape[0]
    n = nu + npr
    b = adj_matrix.shape[1]
    assert nu % _BR == 0 and npr % _BR == 0 and _BR % _S == 0 and _SR % 16 == 0
    nsteps = n // _BR
    nub = nu // _BR

    a2 = (degreeV_matrix[:, None] * adj_matrix *
          jnp.sqrt(degreeE_matrix)[None, :]).astype(jnp.bfloat16)
    gu = gate_user.reshape(1, 1)
    gp = gate_product.reshape(1, 1)

    m1t = pl.pallas_call(
        functools.partial(_p1, nu_blocks=nub),
        grid=(nsteps,),
        in_specs=_sub_specs(b) + [
            pl.BlockSpec((_BR, dim), lambda k: (jnp.minimum(k, nub - 1), 0)),
            pl.BlockSpec((_BR, dim), lambda k: (jnp.maximum(k - nub, 0), 0)),
            pl.BlockSpec((1, 1), lambda k: (0, 0)),
            pl.BlockSpec((1, 1), lambda k: (0, 0)),
        ],
        out_specs=pl.BlockSpec((dim, b), lambda k: (0, 0)),
        out_shape=jax.ShapeDtypeStruct((dim, b), jnp.float32),
    )(*([a2] * _S), users_embedding, product_embedding, gu, gp)

    e1, m2t = pl.pallas_call(
        functools.partial(_p2, nu_blocks=nub),
        grid=(nsteps,),
        in_specs=_sub_specs(b) + [
            pl.BlockSpec((dim, b), lambda k: (0, 0)),
            pl.BlockSpec((1, 1), lambda k: (0, 0)),
            pl.BlockSpec((1, 1), lambda k: (0, 0)),
        ],
        out_specs=[
            pl.BlockSpec((_BR, dim), lambda k: (k, 0)),
            pl.BlockSpec((dim, b), lambda k: (0, 0)),
        ],
        out_shape=[
            jax.ShapeDtypeStruct((n, dim), jnp.float32),
            jax.ShapeDtypeStruct((dim, b), jnp.float32),
        ],
        scratch_shapes=[pltpu.VMEM((b, dim), jnp.bfloat16)],
    )(*([a2] * _S), m1t, gu, gp)

    user_emb, product_emb = pl.pallas_call(
        functools.partial(_p3, nu_blocks=nub),
        grid=(nsteps,),
        in_specs=_sub_specs(b) + [
            pl.BlockSpec((dim, b), lambda k: (0, 0)),
            pl.BlockSpec((_BR, dim), lambda k: (jnp.minimum(k, nub - 1), 0)),
            pl.BlockSpec((_BR, dim), lambda k: (jnp.maximum(k - nub, 0), 0)),
            pl.BlockSpec((_BR, dim), lambda k: (k, 0)),
        ],
        out_specs=[
            pl.BlockSpec((_BR, dim), lambda k: (jnp.minimum(k, nub - 1), 0)),
            pl.BlockSpec((_BR, dim), lambda k: (jnp.maximum(k - nub, 0), 0)),
        ],
        out_shape=[
            jax.ShapeDtypeStruct((nu, dim), jnp.float32),
            jax.ShapeDtypeStruct((npr, dim), jnp.float32),
        ],
        scratch_shapes=[pltpu.VMEM((b, dim), jnp.bfloat16)],
    )(*([a2] * _S), m2t, users_embedding, product_embedding, e1)

    return (user_emb, product_emb)


# speedup vs baseline: 2.9854x; 2.9854x over previous
import functools

import jax
import jax.numpy as jnp
from jax.experimental import pallas as pl
from jax.experimental.pallas import tpu as pltpu

_BR1 = 400
_BR = 2000
_S = 5
_SR = _BR // _S


def _sub_specs(b):
    return [
        pl.BlockSpec((_SR, b), functools.partial(lambda j, k: (_S * k + j, 0), j))
        for j in range(_S)
    ]


def _p1(a_ref, u_ref, p_ref, d_ref, sqe_ref, gu_ref, gp_ref, m1t_ref, abf_ref,
        *, nu_blocks):
    k = pl.program_id(0)

    @pl.when(k == 0)
    def _():
        m1t_ref[...] = jnp.zeros_like(m1t_ref)

    a2 = (d_ref[...] * a_ref[...] * sqe_ref[...]).astype(jnp.bfloat16)
    abf_ref[...] = a2
    is_user = k < nu_blocks
    e0 = jnp.where(is_user, u_ref[...], p_ref[...])
    g = jnp.where(is_user, gu_ref[0, 0], gp_ref[0, 0])
    w = (g * e0).astype(jnp.bfloat16)
    m1t_ref[...] += jax.lax.dot_general(
        w, a2, (((0,), (0,)), ((), ())), preferred_element_type=jnp.float32)


def _p2(*refs, nu_blocks):
    a_refs = refs[:_S]
    m1t_ref, gu_ref, gp_ref, e1_ref, m2t_ref, y1_scr = refs[_S:]
    k = pl.program_id(0)

    @pl.when(k == 0)
    def _():
        m2t_ref[...] = jnp.zeros_like(m2t_ref)
        y1_scr[...] = m1t_ref[...].astype(jnp.bfloat16).T

    g = jnp.where(k < nu_blocks, gu_ref[0, 0], gp_ref[0, 0])
    y1 = y1_scr[...]
    acc = None
    for j in range(_S):
        a = a_refs[j][...]
        t = jax.lax.dot_general(
            a, y1, (((1,), (0,)), ((), ())), preferred_element_type=jnp.float32)
        e1_ref[j * _SR:(j + 1) * _SR] = t
        x = (g * t).astype(jnp.bfloat16)
        prod = jax.lax.dot_general(
            x, a, (((0,), (0,)), ((), ())), preferred_element_type=jnp.float32)
        acc = prod if acc is None else acc + prod
    m2t_ref[...] += acc


def _p3(*refs, nu_blocks):
    a_refs = refs[:_S]
    m2t_ref, u_ref, p_ref, e1_ref, uo_ref, po_ref, y2_scr = refs[_S:]
    k = pl.program_id(0)

    @pl.when(k == 0)
    def _():
        y2_scr[...] = m2t_ref[...].astype(jnp.bfloat16).T

    is_user = k < nu_blocks
    e0 = jnp.where(is_user, u_ref[...], p_ref[...])
    y2 = y2_scr[...]
    for j in range(_S):
        t = jax.lax.dot_general(
            a_refs[j][...], y2, (((1,), (0,)), ((), ())),
            preferred_element_type=jnp.float32)
        sl = slice(j * _SR, (j + 1) * _SR)
        res = (e0[sl] + e1_ref[sl] + t) * (1.0 / 3.0)

        @pl.when(is_user)
        def _(res=res, sl=sl):
            uo_ref[sl] = res

        @pl.when(jnp.logical_not(is_user))
        def _(res=res, sl=sl):
            po_ref[sl] = res


def kernel(users_embedding, product_embedding, adj_matrix, degreeV_matrix,
           degreeE_matrix, gate_user, gate_product):
    nu, dim = users_embedding.shape
    npr = product_embedding.shape[0]
    n = nu + npr
    b = adj_matrix.shape[1]
    assert nu % _BR1 == 0 and npr % _BR1 == 0
    assert nu % _BR == 0 and npr % _BR == 0 and _BR % _S == 0 and _SR % 16 == 0
    nsteps1 = n // _BR1
    nub1 = nu // _BR1
    nsteps = n // _BR
    nub = nu // _BR

    dcol = degreeV_matrix[:, None]
    sqe = jnp.sqrt(degreeE_matrix)[None, :]
    gu = gate_user.reshape(1, 1)
    gp = gate_product.reshape(1, 1)

    m1t, abf = pl.pallas_call(
        functools.partial(_p1, nu_blocks=nub1),
        grid=(nsteps1,),
        in_specs=[
            pl.BlockSpec((_BR1, b), lambda k: (k, 0)),
            pl.BlockSpec((_BR1, dim), lambda k: (jnp.minimum(k, nub1 - 1), 0)),
            pl.BlockSpec((_BR1, dim), lambda k: (jnp.maximum(k - nub1, 0), 0)),
            pl.BlockSpec((_BR1, 1), lambda k: (k, 0)),
            pl.BlockSpec((1, b), lambda k: (0, 0)),
            pl.BlockSpec((1, 1), lambda k: (0, 0)),
            pl.BlockSpec((1, 1), lambda k: (0, 0)),
        ],
        out_specs=[
            pl.BlockSpec((dim, b), lambda k: (0, 0)),
            pl.BlockSpec((_BR1, b), lambda k: (k, 0)),
        ],
        out_shape=[
            jax.ShapeDtypeStruct((dim, b), jnp.float32),
            jax.ShapeDtypeStruct((n, b), jnp.bfloat16),
        ],
    )(adj_matrix, users_embedding, product_embedding, dcol, sqe, gu, gp)

    e1, m2t = pl.pallas_call(
        functools.partial(_p2, nu_blocks=nub),
        grid=(nsteps,),
        in_specs=_sub_specs(b) + [
            pl.BlockSpec((dim, b), lambda k: (0, 0)),
            pl.BlockSpec((1, 1), lambda k: (0, 0)),
            pl.BlockSpec((1, 1), lambda k: (0, 0)),
        ],
        out_specs=[
            pl.BlockSpec((_BR, dim), lambda k: (k, 0)),
            pl.BlockSpec((dim, b), lambda k: (0, 0)),
        ],
        out_shape=[
            jax.ShapeDtypeStruct((n, dim), jnp.float32),
            jax.ShapeDtypeStruct((dim, b), jnp.float32),
        ],
        scratch_shapes=[pltpu.VMEM((b, dim), jnp.bfloat16)],
    )(*([abf] * _S), m1t, gu, gp)

    user_emb, product_emb = pl.pallas_call(
        functools.partial(_p3, nu_blocks=nub),
        grid=(nsteps,),
        in_specs=_sub_specs(b) + [
            pl.BlockSpec((dim, b), lambda k: (0, 0)),
            pl.BlockSpec((_BR, dim), lambda k: (jnp.minimum(k, nub - 1), 0)),
            pl.BlockSpec((_BR, dim), lambda k: (jnp.maximum(k - nub, 0), 0)),
            pl.BlockSpec((_BR, dim), lambda k: (k, 0)),
        ],
        out_specs=[
            pl.BlockSpec((_BR, dim), lambda k: (jnp.minimum(k, nub - 1), 0)),
            pl.BlockSpec((_BR, dim), lambda k: (jnp.maximum(k - nub, 0), 0)),
        ],
        out_shape=[
            jax.ShapeDtypeStruct((nu, dim), jnp.float32),
            jax.ShapeDtypeStruct((npr, dim), jnp.float32),
        ],
        scratch_shapes=[pltpu.VMEM((b, dim), jnp.bfloat16)],
    )(*([abf] * _S), m2t, users_embedding, product_embedding, e1)

    return (user_emb, product_emb)


# speedup vs baseline: 3.0926x; 1.0359x over previous
import functools

import jax
import jax.numpy as jnp
from jax.experimental import pallas as pl
from jax.experimental.pallas import tpu as pltpu

_BR = 1000
_S = 5
_SR = _BR // _S


def _sub_specs(b):
    return [
        pl.BlockSpec((_SR, b), functools.partial(lambda j, k: (_S * k + j, 0), j))
        for j in range(_S)
    ]


def _p1(*refs, nu_blocks):
    a_refs = refs[:_S]
    u_ref, p_ref, d_ref, gu_ref, gp_ref, m1t_ref = refs[_S:]
    k = pl.program_id(0)

    @pl.when(k == 0)
    def _():
        m1t_ref[...] = jnp.zeros_like(m1t_ref)

    is_user = k < nu_blocks
    e0 = jnp.where(is_user, u_ref[...], p_ref[...])
    g = jnp.where(is_user, gu_ref[0, 0], gp_ref[0, 0])
    w = (g * d_ref[...] * e0).astype(jnp.bfloat16)
    acc = None
    for j in range(_S):
        a = a_refs[j][...].astype(jnp.bfloat16)
        prod = jax.lax.dot_general(
            w[j * _SR:(j + 1) * _SR], a, (((0,), (0,)), ((), ())),
            preferred_element_type=jnp.float32)
        acc = prod if acc is None else acc + prod
    m1t_ref[...] += acc


def _p2(*refs, nu_blocks):
    a_refs = refs[:_S]
    m1t_ref, e_ref, d_ref, gu_ref, gp_ref, e1_ref, m2t_ref, y1_scr = refs[_S:]
    k = pl.program_id(0)

    @pl.when(k == 0)
    def _():
        m2t_ref[...] = jnp.zeros_like(m2t_ref)
        y1_scr[...] = (e_ref[...] * m1t_ref[...]).astype(jnp.bfloat16).T

    g = jnp.where(k < nu_blocks, gu_ref[0, 0], gp_ref[0, 0])
    y1 = y1_scr[...]
    acc = None
    for j in range(_S):
        a = a_refs[j][...].astype(jnp.bfloat16)
        t = jax.lax.dot_general(
            a, y1, (((1,), (0,)), ((), ())), preferred_element_type=jnp.float32)
        d = d_ref[j * _SR:(j + 1) * _SR]
        e1_ref[j * _SR:(j + 1) * _SR] = d * t
        x = (g * d * d * t).astype(jnp.bfloat16)
        prod = jax.lax.dot_general(
            x, a, (((0,), (0,)), ((), ())), preferred_element_type=jnp.float32)
        acc = prod if acc is None else acc + prod
    m2t_ref[...] += acc


def _p3(*refs, nu_blocks):
    a_refs = refs[:_S]
    (m2t_ref, e_ref, d_ref, u_ref, p_ref, e1_ref, uo_ref, po_ref,
     y2_scr) = refs[_S:]
    k = pl.program_id(0)

    @pl.when(k == 0)
    def _():
        y2_scr[...] = (e_ref[...] * m2t_ref[...]).astype(jnp.bfloat16).T

    is_user = k < nu_blocks
    e0 = jnp.where(is_user, u_ref[...], p_ref[...])
    y2 = y2_scr[...]
    for j in range(_S):
        a = a_refs[j][...].astype(jnp.bfloat16)
        t = jax.lax.dot_general(
            a, y2, (((1,), (0,)), ((), ())), preferred_element_type=jnp.float32)
        sl = slice(j * _SR, (j + 1) * _SR)
        res = (e0[sl] + e1_ref[sl] + d_ref[sl] * t) * (1.0 / 3.0)

        @pl.when(is_user)
        def _(res=res, sl=sl):
            uo_ref[sl] = res

        @pl.when(jnp.logical_not(is_user))
        def _(res=res, sl=sl):
            po_ref[sl] = res


def kernel(users_embedding, product_embedding, adj_matrix, degreeV_matrix,
           degreeE_matrix, gate_user, gate_product):
    nu, dim = users_embedding.shape
    npr = product_embedding.shape[0]
    n = nu + npr
    b = adj_matrix.shape[1]
    assert nu % _BR == 0 and npr % _BR == 0 and _BR % _S == 0 and _SR % 8 == 0
    nsteps = n // _BR
    nub = nu // _BR

    dcol = degreeV_matrix[:, None]
    erow = degreeE_matrix[None, :]
    gu = gate_user.reshape(1, 1)
    gp = gate_product.reshape(1, 1)

    m1t = pl.pallas_call(
        functools.partial(_p1, nu_blocks=nub),
        grid=(nsteps,),
        in_specs=_sub_specs(b) + [
            pl.BlockSpec((_BR, dim), lambda k: (jnp.minimum(k, nub - 1), 0)),
            pl.BlockSpec((_BR, dim), lambda k: (jnp.maximum(k - nub, 0), 0)),
            pl.BlockSpec((_BR, 1), lambda k: (k, 0)),
            pl.BlockSpec((1, 1), lambda k: (0, 0)),
            pl.BlockSpec((1, 1), lambda k: (0, 0)),
        ],
        out_specs=pl.BlockSpec((dim, b), lambda k: (0, 0)),
        out_shape=jax.ShapeDtypeStruct((dim, b), jnp.float32),
    )(*([adj_matrix] * _S), users_embedding, product_embedding, dcol, gu, gp)

    e1, m2t = pl.pallas_call(
        functools.partial(_p2, nu_blocks=nub),
        grid=(nsteps,),
        in_specs=_sub_specs(b) + [
            pl.BlockSpec((dim, b), lambda k: (0, 0)),
            pl.BlockSpec((1, b), lambda k: (0, 0)),
            pl.BlockSpec((_BR, 1), lambda k: (k, 0)),
            pl.BlockSpec((1, 1), lambda k: (0, 0)),
            pl.BlockSpec((1, 1), lambda k: (0, 0)),
        ],
        out_specs=[
            pl.BlockSpec((_BR, dim), lambda k: (k, 0)),
            pl.BlockSpec((dim, b), lambda k: (0, 0)),
        ],
        out_shape=[
            jax.ShapeDtypeStruct((n, dim), jnp.float32),
            jax.ShapeDtypeStruct((dim, b), jnp.float32),
        ],
        scratch_shapes=[pltpu.VMEM((b, dim), jnp.bfloat16)],
    )(*([adj_matrix] * _S), m1t, erow, dcol, gu, gp)

    user_emb, product_emb = pl.pallas_call(
        functools.partial(_p3, nu_blocks=nub),
        grid=(nsteps,),
        in_specs=_sub_specs(b) + [
            pl.BlockSpec((dim, b), lambda k: (0, 0)),
            pl.BlockSpec((1, b), lambda k: (0, 0)),
            pl.BlockSpec((_BR, 1), lambda k: (k, 0)),
            pl.BlockSpec((_BR, dim), lambda k: (jnp.minimum(k, nub - 1), 0)),
            pl.BlockSpec((_BR, dim), lambda k: (jnp.maximum(k - nub, 0), 0)),
            pl.BlockSpec((_BR, dim), lambda k: (k, 0)),
        ],
        out_specs=[
            pl.BlockSpec((_BR, dim), lambda k: (jnp.minimum(k, nub - 1), 0)),
            pl.BlockSpec((_BR, dim), lambda k: (jnp.maximum(k - nub, 0), 0)),
        ],
        out_shape=[
            jax.ShapeDtypeStruct((nu, dim), jnp.float32),
            jax.ShapeDtypeStruct((npr, dim), jnp.float32),
        ],
        scratch_shapes=[pltpu.VMEM((b, dim), jnp.bfloat16)],
    )(*([adj_matrix] * _S), m2t, erow, dcol, users_embedding,
      product_embedding, e1)

    return (user_emb, product_emb)


# speedup vs baseline: 3.1302x; 1.0122x over previous
import functools

import jax
import jax.numpy as jnp
from jax.experimental import pallas as pl
from jax.experimental.pallas import tpu as pltpu

_BR = 1000


def _body(a_ref, u_ref, p_ref, d_ref, e_ref, gu_ref, gp_ref, uo_ref, po_ref,
          m1t_scr, m2t_scr, e1_scr, y_scr, *, nsteps, nub):
    k = pl.program_id(0)
    kb = jax.lax.rem(k, nsteps)
    is_user = kb < nub
    g = jnp.where(is_user, gu_ref[0, 0], gp_ref[0, 0])
    d = d_ref[...]
    abf = a_ref[...].astype(jnp.bfloat16)

    @pl.when(k == 0)
    def _():
        m1t_scr[...] = jnp.zeros_like(m1t_scr)
        m2t_scr[...] = jnp.zeros_like(m2t_scr)

    @pl.when(k < nsteps)
    def _():
        e0 = jnp.where(is_user, u_ref[...], p_ref[...])
        w = (g * d * e0).astype(jnp.bfloat16)
        m1t_scr[...] += jax.lax.dot_general(
            w, abf, (((0,), (0,)), ((), ())),
            preferred_element_type=jnp.float32)

    @pl.when(k == nsteps)
    def _():
        y_scr[...] = (e_ref[...] * m1t_scr[...]).astype(jnp.bfloat16).T

    @pl.when(jnp.logical_and(k >= nsteps, k < 2 * nsteps))
    def _():
        t = jax.lax.dot_general(
            abf, y_scr[...], (((1,), (0,)), ((), ())),
            preferred_element_type=jnp.float32)
        e1_scr[pl.ds(kb * _BR, _BR), :] = (d * t).astype(jnp.bfloat16)
        x = (g * d * d * t).astype(jnp.bfloat16)
        m2t_scr[...] += jax.lax.dot_general(
            x, abf, (((0,), (0,)), ((), ())),
            preferred_element_type=jnp.float32)

    @pl.when(k == 2 * nsteps)
    def _():
        y_scr[...] = (e_ref[...] * m2t_scr[...]).astype(jnp.bfloat16).T

    @pl.when(k >= 2 * nsteps)
    def _():
        t = jax.lax.dot_general(
            abf, y_scr[...], (((1,), (0,)), ((), ())),
            preferred_element_type=jnp.float32)
        e0 = jnp.where(is_user, u_ref[...], p_ref[...])
        e1 = e1_scr[pl.ds(kb * _BR, _BR), :].astype(jnp.float32)
        res = (e0 + e1 + d * t) * (1.0 / 3.0)

        @pl.when(is_user)
        def _():
            uo_ref[...] = res

        @pl.when(jnp.logical_not(is_user))
        def _():
            po_ref[...] = res


def kernel(users_embedding, product_embedding, adj_matrix, degreeV_matrix,
           degreeE_matrix, gate_user, gate_product):
    nu, dim = users_embedding.shape
    npr = product_embedding.shape[0]
    n = nu + npr
    b = adj_matrix.shape[1]
    assert nu % _BR == 0 and npr % _BR == 0
    nsteps = n // _BR
    nub = nu // _BR
    npb = npr // _BR

    dcol = degreeV_matrix[:, None]
    erow = degreeE_matrix[None, :]
    gu = gate_user.reshape(1, 1)
    gp = gate_product.reshape(1, 1)

    def _amap(k):
        return (jax.lax.rem(k, nsteps), 0)

    def _umap(k):
        kb = jax.lax.rem(k, nsteps)
        return (jnp.minimum(kb, nub - 1), 0)

    def _pmap(k):
        kb = jax.lax.rem(k, nsteps)
        return (jnp.clip(kb - nub, 0, npb - 1), 0)

    def _uomap(k):
        return (jnp.clip(k - 2 * nsteps, 0, nub - 1), 0)

    def _pomap(k):
        return (jnp.clip(k - 2 * nsteps - nub, 0, npb - 1), 0)

    user_emb, product_emb = pl.pallas_call(
        functools.partial(_body, nsteps=nsteps, nub=nub),
        grid=(3 * nsteps,),
        in_specs=[
            pl.BlockSpec((_BR, b), _amap),
            pl.BlockSpec((_BR, dim), _umap),
            pl.BlockSpec((_BR, dim), _pmap),
            pl.BlockSpec((_BR, 1), _amap),
            pl.BlockSpec((1, b), lambda k: (0, 0)),
            pl.BlockSpec((1, 1), lambda k: (0, 0)),
            pl.BlockSpec((1, 1), lambda k: (0, 0)),
        ],
        out_specs=[
            pl.BlockSpec((_BR, dim), _uomap),
            pl.BlockSpec((_BR, dim), _pomap),
        ],
        out_shape=[
            jax.ShapeDtypeStruct((nu, dim), jnp.float32),
            jax.ShapeDtypeStruct((npr, dim), jnp.float32),
        ],
        scratch_shapes=[
            pltpu.VMEM((dim, b), jnp.float32),
            pltpu.VMEM((dim, b), jnp.float32),
            pltpu.VMEM((n, dim), jnp.bfloat16),
            pltpu.VMEM((b, dim), jnp.bfloat16),
        ],
        compiler_params=pltpu.CompilerParams(
            vmem_limit_bytes=128 * 1024 * 1024),
    )(adj_matrix, users_embedding, product_embedding, dcol, erow, gu, gp)

    return (user_emb, product_emb)
